# baseline (speedup 1.0000x reference)
import jax
import jax.numpy as jnp
from jax.experimental import pallas as pl
from jax.experimental.pallas import tpu as pltpu


_CHUNKS = 8


def _body(pos_ref, hid_ref, mask_ref, emb_ref, out_ref, mask_out_ref, sem):
    B, S = hid_ref.shape[0], hid_ref.shape[1]
    NCT = emb_ref.shape[1]
    p = pos_ref[0]
    csz = S // _CHUNKS

    copies = []
    for b in range(B):
        for c in range(_CHUNKS):
            cp = pltpu.make_async_copy(
                hid_ref.at[b, c * csz:(c + 1) * csz],
                out_ref.at[b, NCT + c * csz:NCT + (c + 1) * csz],
                sem.at[b * _CHUNKS + c],
            )
            cp.start()
            copies.append(cp)
    ctx_copies = []
    for b in range(B):
        cp = pltpu.make_async_copy(
            emb_ref.at[p], out_ref.at[b, :NCT], sem.at[B * _CHUNKS + b]
        )
        cp.start()
        ctx_copies.append(cp)

    mask_out_ref[:, :NCT] = jnp.ones((B, NCT), mask_out_ref.dtype)
    mask_out_ref[:, NCT:] = mask_ref[:, :]

    for cp in copies:
        cp.wait()
    for cp in ctx_copies:
        cp.wait()


def kernel(hidden_states, attention_mask, embeddings, position):
    B, S, D = hidden_states.shape
    _, NCT, _ = embeddings.shape
    pos = jnp.asarray(position, jnp.int32).reshape((1,))
    hid4 = hidden_states.reshape(B, S, 8, D // 8)
    emb4 = embeddings.reshape(embeddings.shape[0], NCT, 8, D // 8)

    out_hid, out_mask = pl.pallas_call(
        _body,
        in_specs=[
            pl.BlockSpec(memory_space=pltpu.SMEM),
            pl.BlockSpec(memory_space=pl.ANY),
            pl.BlockSpec(memory_space=pltpu.VMEM),
            pl.BlockSpec(memory_space=pl.ANY),
        ],
        out_specs=[
            pl.BlockSpec(memory_space=pl.ANY),
            pl.BlockSpec(memory_space=pltpu.VMEM),
        ],
        out_shape=[
            jax.ShapeDtypeStruct((B, NCT + S, 8, D // 8), hidden_states.dtype),
            jax.ShapeDtypeStruct((B, NCT + S), attention_mask.dtype),
        ],
        scratch_shapes=[pltpu.SemaphoreType.DMA((B * _CHUNKS + B,))],
    )(pos, hid4, attention_mask, emb4)
    return (out_hid.reshape(B, NCT + S, D), out_mask)


# speedup vs baseline: 8.2875x; 8.2875x over previous
import jax
import jax.numpy as jnp
from jax.experimental import pallas as pl
from jax.experimental.pallas import tpu as pltpu

S_BLK = 256


def _body(pos_ref, hid_ref, mask_ref, emb_ref, out_ref, mask_out_ref):
    nct = emb_ref.shape[1]
    k = pl.program_id(1)
    out_ref[0, pl.ds(nct + k * S_BLK, S_BLK)] = hid_ref[0]

    @pl.when(k == 0)
    def _():
        out_ref[0, :nct] = emb_ref[0]
        mask_out_ref[0, 0, :nct] = jnp.ones((nct,), mask_out_ref.dtype)
        mask_out_ref[0, 0, nct:] = mask_ref[0, 0]


def kernel(hidden_states, attention_mask, embeddings, position):
    B, S, D = hidden_states.shape
    _, NCT, _ = embeddings.shape
    pos = jnp.asarray(position, jnp.int32).reshape((1,))
    ns = S // S_BLK
    hid4 = hidden_states.reshape(B, S, 8, D // 8)
    emb4 = embeddings.reshape(embeddings.shape[0], NCT, 8, D // 8)
    mask3 = attention_mask.reshape(B, 1, S)

    grid_spec = pltpu.PrefetchScalarGridSpec(
        num_scalar_prefetch=1,
        grid=(B, ns),
        in_specs=[
            pl.BlockSpec((1, S_BLK, 8, D // 8), lambda b, k, p: (b, k, 0, 0)),
            pl.BlockSpec((1, 1, S), lambda b, k, p: (b, 0, 0)),
            pl.BlockSpec((1, NCT, 8, D // 8), lambda b, k, p: (p[0], 0, 0, 0)),
        ],
        out_specs=[
            pl.BlockSpec((1, NCT + S, 8, D // 8), lambda b, k, p: (b, 0, 0, 0)),
            pl.BlockSpec((1, 1, NCT + S), lambda b, k, p: (b, 0, 0)),
        ],
    )

    out_hid, out_mask = pl.pallas_call(
        _body,
        grid_spec=grid_spec,
        out_shape=[
            jax.ShapeDtypeStruct((B, NCT + S, 8, D // 8), hidden_states.dtype),
            jax.ShapeDtypeStruct((B, 1, NCT + S), attention_mask.dtype),
        ],
    )(pos, hid4, mask3, emb4)
    return (out_hid.reshape(B, NCT + S, D), out_mask.reshape(B, NCT + S))
